# baseline (speedup 1.0000x reference)
import functools

import jax
import jax.numpy as jnp
from jax import lax
from jax.experimental import pallas as pl
from jax.experimental.pallas import tpu as pltpu
from jax.experimental.pallas import tpu_sc as plsc

_N = 10000
_D = 128
_E_MESH = 320000
_E_WORLD = 160000
_NW = 32
_CH = 256


def _precompute_tables(nf, w_stack):
    blk = 2000
    grid = _N // blk

    def body(nf_ref, w_ref, oa_m, ob_m, oa_w, ob_w):
        x = nf_ref[...].astype(jnp.bfloat16)
        for j, o_ref in enumerate((oa_m, ob_m, oa_w, ob_w)):
            o_ref[...] = jnp.dot(x, w_ref[j].astype(jnp.bfloat16),
                                 preferred_element_type=jnp.float32)

    out = pl.pallas_call(
        body,
        grid=(grid,),
        in_specs=[
            pl.BlockSpec((blk, _D), lambda i: (i, 0)),
            pl.BlockSpec((4, _D, _D), lambda i: (0, 0, 0)),
        ],
        out_specs=[pl.BlockSpec((blk, _D), lambda i: (i, 0))] * 4,
        out_shape=[jax.ShapeDtypeStruct((_N, _D), jnp.float32)] * 4,
    )(nf, w_stack)
    return out


_GCH = 128


def _sc_gather(ta, tb, sidx, ridx, n_edges):
    mesh = plsc.VectorSubcoreMesh(core_axis_name="c", subcore_axis_name="s")

    @functools.partial(
        pl.kernel,
        out_type=jax.ShapeDtypeStruct((n_edges, _D), jnp.float32),
        mesh=mesh,
        scratch_types=[
            pltpu.VMEM((_GCH,), jnp.int32),
            pltpu.VMEM((_GCH,), jnp.int32),
            pltpu.VMEM((_GCH, _D), jnp.float32),
            pltpu.VMEM((_GCH, _D), jnp.float32),
            pltpu.VMEM_SHARED((_N, _D), jnp.float32),
            pltpu.SemaphoreType.DMA,
        ],
    )
    def k(ta_r, tb_r, si_r, ri_r, g_out,
          idxs, idxr, buf_a, buf_b, stab, sem):
        sub = lax.axis_index("s")
        wid = lax.axis_index("c") * 16 + sub
        zrows, nzchunks = 80, _N // 80

        def stage(step, carry):
            cid = sub + step * 16

            @pl.when(cid < nzchunks)
            def _():
                sl = pl.ds(cid * zrows, zrows)
                pltpu.sync_copy(ta_r.at[sl], stab.at[sl])

            return carry

        lax.fori_loop(0, (nzchunks + 15) // 16, stage, None)
        plsc.subcore_barrier()

        nchunks = n_edges // _GCH
        niter = (nchunks + _NW - 1) // _NW

        def body(step, carry):
            cid = wid + step * _NW

            @pl.when(cid < nchunks)
            def _():
                base = cid * _GCH
                pltpu.sync_copy(si_r.at[pl.ds(base, _GCH)], idxs)
                pltpu.sync_copy(ri_r.at[pl.ds(base, _GCH)], idxr)
                ca = pltpu.async_copy(stab.at[idxs], buf_a, sem)
                ca.wait()
                cb = pltpu.async_copy(tb_r.at[idxr], buf_b, sem)
                cb.wait()

                def rbody(r, carry2):
                    for g in range(_D // 16):
                        sl = pl.ds(g * 16, 16)
                        buf_a[r, sl] = buf_a[r, sl] + buf_b[r, sl]
                    return carry2

                lax.fori_loop(0, _GCH, rbody, None)
                pltpu.sync_copy(buf_a, g_out.at[pl.ds(base, _GCH)])

            return carry

        lax.fori_loop(0, niter, body, None)

    return k(ta, tb, sidx, ridx)


def _edge_mlp(gsum, ef, w1e, b1, w2, b2, ln_g, ln_b):
    n_edges = gsum.shape[0]
    blk = 640
    grid = n_edges // blk

    def body(gs_r, ef_r, w1_r, b1_r, w2_r, b2_r, g_r, be_r,
             new_r, out_r):
        ef = ef_r[...]
        pre = (gs_r[...]
               + jnp.dot(ef.astype(jnp.bfloat16),
                         w1_r[...].astype(jnp.bfloat16),
                         preferred_element_type=jnp.float32)
               + b1_r[...])
        h = jnp.maximum(pre, 0.0)
        o = jnp.dot(h.astype(jnp.bfloat16), w2_r[...].astype(jnp.bfloat16),
                    preferred_element_type=jnp.float32) + b2_r[...]
        mu = jnp.mean(o, axis=-1, keepdims=True)
        var = jnp.mean((o - mu) ** 2, axis=-1, keepdims=True)
        ln = (o - mu) * lax.rsqrt(var + 1e-5) * g_r[...] + be_r[...]
        new_r[...] = ln
        out_r[...] = ln + ef

    row = pl.BlockSpec((blk, _D), lambda i: (i, 0))
    full = pl.BlockSpec((_D, _D), lambda i: (0, 0))
    vec = pl.BlockSpec((1, _D), lambda i: (0, 0))
    return pl.pallas_call(
        body,
        grid=(grid,),
        in_specs=[row, row, full, vec, full, vec, vec, vec],
        out_specs=[row, row],
        out_shape=[jax.ShapeDtypeStruct((n_edges, _D), jnp.float32)] * 2,
    )(gsum, ef, w1e, b1.reshape(1, _D), w2, b2.reshape(1, _D),
      ln_g.reshape(1, _D), ln_b.reshape(1, _D))


def _sc_scatter(new_e, ridx, n_edges):
    mesh = plsc.VectorSubcoreMesh(core_axis_name="c", subcore_axis_name="s")
    zrows = 80
    nzchunks = _N // zrows

    @functools.partial(
        pl.kernel,
        out_type=jax.ShapeDtypeStruct((2 * _N, _D), jnp.float32),
        mesh=mesh,
        scratch_types=[
            pltpu.VMEM((zrows, _D), jnp.float32),
            pltpu.VMEM((2, 128), jnp.int32),
            pltpu.VMEM((_CH, _D), jnp.float32),
            pltpu.VMEM_SHARED((_N, _D), jnp.float32),
            pltpu.SemaphoreType.DMA,
        ],
    )
    def k(src, idx, out, zbuf, idx_v, rows_v, acc, sem):
        core = lax.axis_index("c")
        sub = lax.axis_index("s")
        wid = core * 16 + sub

        zero16 = jnp.zeros((16,), jnp.float32)

        def zbody(i, carry):
            for j in range(_D // 16):
                zbuf[i, pl.ds(j * 16, 16)] = zero16
            return carry

        lax.fori_loop(0, zrows, zbody, None)

        def zero_acc(step, carry):
            cid = sub + step * 16

            @pl.when(cid < nzchunks)
            def _():
                pltpu.sync_copy(zbuf, acc.at[pl.ds(cid * zrows, zrows)])

            return carry

        lax.fori_loop(0, (nzchunks + 15) // 16, zero_acc, None)
        plsc.subcore_barrier()

        nchunks = n_edges // _CH
        niter = (nchunks + _NW - 1) // _NW

        def body(step, carry):
            cid = wid + step * _NW

            @pl.when(cid < nchunks)
            def _():
                base = cid * _CH
                pltpu.sync_copy(idx.at[pl.ds(base, 128)], idx_v.at[0])
                pltpu.sync_copy(idx.at[pl.ds(base + 128, 128)], idx_v.at[1])
                pltpu.sync_copy(src.at[pl.ds(base, _CH)], rows_v)
                pltpu.sync_copy(rows_v.at[pl.ds(0, 128)],
                                acc.at[idx_v.at[0]], add=True)
                pltpu.sync_copy(rows_v.at[pl.ds(128, 128)],
                                acc.at[idx_v.at[1]], add=True)

            return carry

        lax.fori_loop(0, niter, body, None)
        plsc.subcore_barrier()

        def writeout(step, carry):
            cid = sub + step * 16

            @pl.when(cid < nzchunks)
            def _():
                pltpu.sync_copy(
                    acc.at[pl.ds(cid * zrows, zrows)],
                    out.at[pl.ds(core * _N + cid * zrows, zrows)])

            return carry

        lax.fori_loop(0, (nzchunks + 15) // 16, writeout, None)
        plsc.subcore_barrier()

    return k(new_e, ridx)


def _node_mlp(nf, am0, am1, aw0, aw1, w_stack, b1, w2, b2, ln_g, ln_b):
    blk = 1000
    grid = _N // blk

    def body(nf_r, am0_r, am1_r, aw0_r, aw1_r, w_r, b1_r, w2_r, b2_r,
             g_r, be_r, out_r):
        nfx = nf_r[...]
        agg_m = am0_r[...] + am1_r[...]
        agg_w = aw0_r[...] + aw1_r[...]
        w16 = w_r[...].astype(jnp.bfloat16)
        pre = (jnp.dot(nfx.astype(jnp.bfloat16), w16[0],
                       preferred_element_type=jnp.float32)
               + jnp.dot(agg_m.astype(jnp.bfloat16), w16[1],
                         preferred_element_type=jnp.float32)
               + jnp.dot(agg_w.astype(jnp.bfloat16), w16[2],
                         preferred_element_type=jnp.float32)
               + b1_r[...])
        h = jnp.maximum(pre, 0.0)
        o = jnp.dot(h.astype(jnp.bfloat16), w2_r[...].astype(jnp.bfloat16),
                    preferred_element_type=jnp.float32) + b2_r[...]
        mu = jnp.mean(o, axis=-1, keepdims=True)
        var = jnp.mean((o - mu) ** 2, axis=-1, keepdims=True)
        ln = (o - mu) * lax.rsqrt(var + 1e-5) * g_r[...] + be_r[...]
        out_r[...] = ln + nfx

    row = pl.BlockSpec((blk, _D), lambda i: (i, 0))
    full = pl.BlockSpec((_D, _D), lambda i: (0, 0))
    vec = pl.BlockSpec((1, _D), lambda i: (0, 0))
    return pl.pallas_call(
        body,
        grid=(grid,),
        in_specs=[row, row, row, row, row,
                  pl.BlockSpec((3, _D, _D), lambda i: (0, 0, 0)),
                  vec, full, vec, vec, vec],
        out_specs=row,
        out_shape=jax.ShapeDtypeStruct((_N, _D), jnp.float32),
    )(nf, am0, am1, aw0, aw1, w_stack, b1.reshape(1, _D), w2,
      b2.reshape(1, _D), ln_g.reshape(1, _D), ln_b.reshape(1, _D))


def kernel(node_features, mesh_edge_features, mesh_senders, mesh_receivers,
           world_edge_features, world_senders, world_receivers, params):
    pm, pw, pn = params["mesh_edge"], params["world_edge"], params["node"]

    w_gather = jnp.stack([pm["W1"][:_D], pm["W1"][_D:2 * _D],
                          pw["W1"][:_D], pw["W1"][_D:2 * _D]])
    tam, tbm, taw, tbw = _precompute_tables(node_features, w_gather)

    ms = mesh_senders.astype(jnp.int32)
    mr = mesh_receivers.astype(jnp.int32)
    ws = world_senders.astype(jnp.int32)
    wr = world_receivers.astype(jnp.int32)

    gm = _sc_gather(tam, tbm, ms, mr, _E_MESH)
    gw = _sc_gather(taw, tbw, ws, wr, _E_WORLD)

    new_m, out_m = _edge_mlp(gm, mesh_edge_features,
                             pm["W1"][2 * _D:], pm["b1"], pm["W2"], pm["b2"],
                             pm["ln_g"], pm["ln_b"])
    new_w, out_w = _edge_mlp(gw, world_edge_features,
                             pw["W1"][2 * _D:], pw["b1"], pw["W2"], pw["b2"],
                             pw["ln_g"], pw["ln_b"])

    aggm2 = _sc_scatter(new_m, mr, _E_MESH)
    aggw2 = _sc_scatter(new_w, wr, _E_WORLD)

    w_node = jnp.stack([pn["W1"][:_D], pn["W1"][_D:2 * _D], pn["W1"][2 * _D:]])
    new_nodes = _node_mlp(node_features,
                          aggm2[:_N], aggm2[_N:], aggw2[:_N], aggw2[_N:],
                          w_node, pn["b1"], pn["W2"], pn["b2"],
                          pn["ln_g"], pn["ln_b"])
    return new_nodes, out_m, out_w


# speedup vs baseline: 1.3012x; 1.3012x over previous
import functools

import jax
import jax.numpy as jnp
from jax import lax
from jax.experimental import pallas as pl
from jax.experimental.pallas import tpu as pltpu
from jax.experimental.pallas import tpu_sc as plsc

_N = 10000
_D = 128
_E_MESH = 320000
_E_WORLD = 160000
_NW = 32
_CH = 256


def _precompute_tables(nf, w_stack):
    blk = 2000
    grid = _N // blk

    def body(nf_ref, w_ref, oa_m, ob_m, oa_w, ob_w):
        x = nf_ref[...].astype(jnp.bfloat16)
        for j, o_ref in enumerate((oa_m, ob_m, oa_w, ob_w)):
            o_ref[...] = jnp.dot(x, w_ref[j].astype(jnp.bfloat16),
                                 preferred_element_type=jnp.float32)

    out = pl.pallas_call(
        body,
        grid=(grid,),
        in_specs=[
            pl.BlockSpec((blk, _D), lambda i: (i, 0)),
            pl.BlockSpec((4, _D, _D), lambda i: (0, 0, 0)),
        ],
        out_specs=[pl.BlockSpec((blk, _D), lambda i: (i, 0))] * 4,
        out_shape=[jax.ShapeDtypeStruct((_N, _D), jnp.float32)] * 4,
    )(nf, w_stack)
    return out


_GCH = 64


def _sc_gather(ta, tb, sidx, ridx, n_edges):
    mesh = plsc.VectorSubcoreMesh(core_axis_name="c", subcore_axis_name="s")
    nchunks = n_edges // _GCH
    niter = (nchunks + _NW - 1) // _NW
    npairs = (niter + 1) // 2

    @functools.partial(
        pl.kernel,
        out_type=jax.ShapeDtypeStruct((n_edges, _D), jnp.float32),
        mesh=mesh,
        scratch_types=[
            pltpu.VMEM((2, _GCH), jnp.int32),
            pltpu.VMEM((2, _GCH), jnp.int32),
            pltpu.VMEM((2, _GCH, _D), jnp.float32),
            pltpu.VMEM((2, _GCH, _D), jnp.float32),
            pltpu.VMEM_SHARED((_N, _D), jnp.float32),
            pltpu.SemaphoreType.DMA,
            pltpu.SemaphoreType.DMA,
            pltpu.SemaphoreType.DMA,
        ],
    )
    def k(ta_r, tb_r, si_r, ri_r, g_out, idxs, idxr, buf_a, buf_b, stab,
          sem_i, sem_g, sem_o):
        sub = lax.axis_index("s")
        wid = lax.axis_index("c") * 16 + sub
        zrows, nzchunks = 80, _N // 80

        def stage(step, carry):
            cid = sub + step * 16

            @pl.when(cid < nzchunks)
            def _():
                sl = pl.ds(cid * zrows, zrows)
                pltpu.sync_copy(ta_r.at[sl], stab.at[sl])

            return carry

        lax.fori_loop(0, (nzchunks + 15) // 16, stage, None)
        plsc.subcore_barrier()

        def fetch(cid, slot):
            ca = pltpu.async_copy(stab.at[idxs.at[slot]], buf_a.at[slot],
                                  sem_g)
            ca.wait()
            pltpu.async_copy(tb_r.at[idxr.at[slot]], buf_b.at[slot], sem_g)

        @pl.when(wid < nchunks)
        def _():
            base = wid * _GCH
            pltpu.sync_copy(si_r.at[pl.ds(base, _GCH)], idxs.at[0])
            pltpu.sync_copy(ri_r.at[pl.ds(base, _GCH)], idxr.at[0])
            fetch(wid, 0)

        def body(step, carry):
            for u in range(2):
                s, o = (0, 1) if u == 0 else (1, 0)
                cid = wid + (2 * step + u) * _NW
                nxt = cid + _NW

                @pl.when(cid < nchunks)
                def _(cid=cid, nxt=nxt, s=s, o=o):
                    @pl.when(nxt < nchunks)
                    def _():
                        nb = nxt * _GCH
                        pltpu.async_copy(si_r.at[pl.ds(nb, _GCH)],
                                         idxs.at[o], sem_i)
                        pltpu.async_copy(ri_r.at[pl.ds(nb, _GCH)],
                                         idxr.at[o], sem_i)

                    pltpu.make_async_copy(tb_r.at[pl.ds(0, _GCH)],
                                          buf_b.at[s], sem_g).wait()

                    def rbody(r, c2):
                        for g in range(_D // 16):
                            sl = pl.ds(g * 16, 16)
                            buf_a[s, r, sl] = buf_a[s, r, sl] + buf_b[s, r, sl]
                        return c2

                    lax.fori_loop(0, _GCH, rbody, None)

                    @pl.when(nxt < nchunks)
                    def _():
                        pltpu.make_async_copy(si_r.at[pl.ds(0, _GCH)],
                                              idxs.at[o], sem_i).wait()
                        pltpu.make_async_copy(ri_r.at[pl.ds(0, _GCH)],
                                              idxr.at[o], sem_i).wait()

                        @pl.when(cid > wid)
                        def _():
                            pltpu.make_async_copy(
                                buf_a.at[o], g_out.at[pl.ds(0, _GCH)],
                                sem_o).wait()

                        fetch(nxt, o)

                    pltpu.async_copy(buf_a.at[s],
                                     g_out.at[pl.ds(cid * _GCH, _GCH)], sem_o)

            return carry

        lax.fori_loop(0, npairs, body, None)

        @pl.when(wid < nchunks)
        def _():
            pltpu.make_async_copy(buf_a.at[0], g_out.at[pl.ds(0, _GCH)],
                                  sem_o).wait()

        @pl.when(wid + _NW < nchunks)
        def _():
            pltpu.make_async_copy(buf_a.at[1], g_out.at[pl.ds(0, _GCH)],
                                  sem_o).wait()

    return k(ta, tb, sidx, ridx)


def _edge_mlp(gsum, ef, w1e, b1, w2, b2, ln_g, ln_b):
    n_edges = gsum.shape[0]
    blk = 2000
    grid = n_edges // blk

    def body(gs_r, ef_r, w1_r, b1_r, w2_r, b2_r, g_r, be_r,
             new_r, out_r):
        ef = ef_r[...]
        pre = (gs_r[...]
               + jnp.dot(ef.astype(jnp.bfloat16),
                         w1_r[...].astype(jnp.bfloat16),
                         preferred_element_type=jnp.float32)
               + b1_r[...])
        h = jnp.maximum(pre, 0.0)
        o = jnp.dot(h.astype(jnp.bfloat16), w2_r[...].astype(jnp.bfloat16),
                    preferred_element_type=jnp.float32) + b2_r[...]
        mu = jnp.mean(o, axis=-1, keepdims=True)
        var = jnp.mean((o - mu) ** 2, axis=-1, keepdims=True)
        ln = (o - mu) * lax.rsqrt(var + 1e-5) * g_r[...] + be_r[...]
        new_r[...] = ln
        out_r[...] = ln + ef

    row = pl.BlockSpec((blk, _D), lambda i: (i, 0))
    full = pl.BlockSpec((_D, _D), lambda i: (0, 0))
    vec = pl.BlockSpec((1, _D), lambda i: (0, 0))
    return pl.pallas_call(
        body,
        grid=(grid,),
        in_specs=[row, row, full, vec, full, vec, vec, vec],
        out_specs=[row, row],
        out_shape=[jax.ShapeDtypeStruct((n_edges, _D), jnp.float32)] * 2,
    )(gsum, ef, w1e, b1.reshape(1, _D), w2, b2.reshape(1, _D),
      ln_g.reshape(1, _D), ln_b.reshape(1, _D))


def _sc_scatter(new_e, ridx, n_edges):
    mesh = plsc.VectorSubcoreMesh(core_axis_name="c", subcore_axis_name="s")
    zrows = 80
    nzchunks = _N // zrows

    @functools.partial(
        pl.kernel,
        out_type=jax.ShapeDtypeStruct((2 * _N, _D), jnp.float32),
        mesh=mesh,
        scratch_types=[
            pltpu.VMEM((zrows, _D), jnp.float32),
            pltpu.VMEM((2, 128), jnp.int32),
            pltpu.VMEM((_CH, _D), jnp.float32),
            pltpu.VMEM_SHARED((_N, _D), jnp.float32),
            pltpu.SemaphoreType.DMA,
        ],
    )
    def k(src, idx, out, zbuf, idx_v, rows_v, acc, sem):
        core = lax.axis_index("c")
        sub = lax.axis_index("s")
        wid = core * 16 + sub

        zero16 = jnp.zeros((16,), jnp.float32)

        def zbody(i, carry):
            for j in range(_D // 16):
                zbuf[i, pl.ds(j * 16, 16)] = zero16
            return carry

        lax.fori_loop(0, zrows, zbody, None)

        def zero_acc(step, carry):
            cid = sub + step * 16

            @pl.when(cid < nzchunks)
            def _():
                pltpu.sync_copy(zbuf, acc.at[pl.ds(cid * zrows, zrows)])

            return carry

        lax.fori_loop(0, (nzchunks + 15) // 16, zero_acc, None)
        plsc.subcore_barrier()

        nchunks = n_edges // _CH
        niter = (nchunks + _NW - 1) // _NW

        def body(step, carry):
            cid = wid + step * _NW

            @pl.when(cid < nchunks)
            def _():
                base = cid * _CH
                pltpu.sync_copy(idx.at[pl.ds(base, 128)], idx_v.at[0])
                pltpu.sync_copy(idx.at[pl.ds(base + 128, 128)], idx_v.at[1])
                pltpu.sync_copy(src.at[pl.ds(base, _CH)], rows_v)
                pltpu.sync_copy(rows_v.at[pl.ds(0, 128)],
                                acc.at[idx_v.at[0]], add=True)
                pltpu.sync_copy(rows_v.at[pl.ds(128, 128)],
                                acc.at[idx_v.at[1]], add=True)

            return carry

        lax.fori_loop(0, niter, body, None)
        plsc.subcore_barrier()

        def writeout(step, carry):
            cid = sub + step * 16

            @pl.when(cid < nzchunks)
            def _():
                pltpu.sync_copy(
                    acc.at[pl.ds(cid * zrows, zrows)],
                    out.at[pl.ds(core * _N + cid * zrows, zrows)])

            return carry

        lax.fori_loop(0, (nzchunks + 15) // 16, writeout, None)
        plsc.subcore_barrier()

    return k(new_e, ridx)


def _node_mlp(nf, am0, am1, aw0, aw1, w_stack, b1, w2, b2, ln_g, ln_b):
    blk = 1000
    grid = _N // blk

    def body(nf_r, am0_r, am1_r, aw0_r, aw1_r, w_r, b1_r, w2_r, b2_r,
             g_r, be_r, out_r):
        nfx = nf_r[...]
        agg_m = am0_r[...] + am1_r[...]
        agg_w = aw0_r[...] + aw1_r[...]
        w16 = w_r[...].astype(jnp.bfloat16)
        pre = (jnp.dot(nfx.astype(jnp.bfloat16), w16[0],
                       preferred_element_type=jnp.float32)
               + jnp.dot(agg_m.astype(jnp.bfloat16), w16[1],
                         preferred_element_type=jnp.float32)
               + jnp.dot(agg_w.astype(jnp.bfloat16), w16[2],
                         preferred_element_type=jnp.float32)
               + b1_r[...])
        h = jnp.maximum(pre, 0.0)
        o = jnp.dot(h.astype(jnp.bfloat16), w2_r[...].astype(jnp.bfloat16),
                    preferred_element_type=jnp.float32) + b2_r[...]
        mu = jnp.mean(o, axis=-1, keepdims=True)
        var = jnp.mean((o - mu) ** 2, axis=-1, keepdims=True)
        ln = (o - mu) * lax.rsqrt(var + 1e-5) * g_r[...] + be_r[...]
        out_r[...] = ln + nfx

    row = pl.BlockSpec((blk, _D), lambda i: (i, 0))
    full = pl.BlockSpec((_D, _D), lambda i: (0, 0))
    vec = pl.BlockSpec((1, _D), lambda i: (0, 0))
    return pl.pallas_call(
        body,
        grid=(grid,),
        in_specs=[row, row, row, row, row,
                  pl.BlockSpec((3, _D, _D), lambda i: (0, 0, 0)),
                  vec, full, vec, vec, vec],
        out_specs=row,
        out_shape=jax.ShapeDtypeStruct((_N, _D), jnp.float32),
    )(nf, am0, am1, aw0, aw1, w_stack, b1.reshape(1, _D), w2,
      b2.reshape(1, _D), ln_g.reshape(1, _D), ln_b.reshape(1, _D))


def kernel(node_features, mesh_edge_features, mesh_senders, mesh_receivers,
           world_edge_features, world_senders, world_receivers, params):
    pm, pw, pn = params["mesh_edge"], params["world_edge"], params["node"]

    w_gather = jnp.stack([pm["W1"][:_D], pm["W1"][_D:2 * _D],
                          pw["W1"][:_D], pw["W1"][_D:2 * _D]])
    tam, tbm, taw, tbw = _precompute_tables(node_features, w_gather)

    ms = mesh_senders.astype(jnp.int32)
    mr = mesh_receivers.astype(jnp.int32)
    ws = world_senders.astype(jnp.int32)
    wr = world_receivers.astype(jnp.int32)

    gm = _sc_gather(tam, tbm, ms, mr, _E_MESH)
    gw = _sc_gather(taw, tbw, ws, wr, _E_WORLD)

    new_m, out_m = _edge_mlp(gm, mesh_edge_features,
                             pm["W1"][2 * _D:], pm["b1"], pm["W2"], pm["b2"],
                             pm["ln_g"], pm["ln_b"])
    new_w, out_w = _edge_mlp(gw, world_edge_features,
                             pw["W1"][2 * _D:], pw["b1"], pw["W2"], pw["b2"],
                             pw["ln_g"], pw["ln_b"])

    aggm2 = _sc_scatter(new_m, mr, _E_MESH)
    aggw2 = _sc_scatter(new_w, wr, _E_WORLD)

    w_node = jnp.stack([pn["W1"][:_D], pn["W1"][_D:2 * _D], pn["W1"][2 * _D:]])
    new_nodes = _node_mlp(node_features,
                          aggm2[:_N], aggm2[_N:], aggw2[:_N], aggw2[_N:],
                          w_node, pn["b1"], pn["W2"], pn["b2"],
                          pn["ln_g"], pn["ln_b"])
    return new_nodes, out_m, out_w


# speedup vs baseline: 1.4217x; 1.0926x over previous
import functools

import jax
import jax.numpy as jnp
from jax import lax
from jax.experimental import pallas as pl
from jax.experimental.pallas import tpu as pltpu
from jax.experimental.pallas import tpu_sc as plsc

_N = 10000
_D = 128
_E_MESH = 320000
_E_WORLD = 160000
_NW = 32
_CH = 256


def _precompute_tables(nf, w_stack):
    blk = 2000
    grid = _N // blk

    def body(nf_ref, w_ref, oa_m, ob_m, oa_w, ob_w):
        x = nf_ref[...].astype(jnp.bfloat16)
        for j, o_ref in enumerate((oa_m, ob_m, oa_w, ob_w)):
            o_ref[...] = jnp.dot(x, w_ref[j].astype(jnp.bfloat16),
                                 preferred_element_type=jnp.float32)

    out = pl.pallas_call(
        body,
        grid=(grid,),
        in_specs=[
            pl.BlockSpec((blk, _D), lambda i: (i, 0)),
            pl.BlockSpec((4, _D, _D), lambda i: (0, 0, 0)),
        ],
        out_specs=[pl.BlockSpec((blk, _D), lambda i: (i, 0))] * 4,
        out_shape=[jax.ShapeDtypeStruct((_N, _D), jnp.float32)] * 4,
    )(nf, w_stack)
    return out


_GCH = 64


def _sc_gather(ta, tb, sidx, ridx, n_edges):
    mesh = plsc.VectorSubcoreMesh(core_axis_name="c", subcore_axis_name="s")
    nchunks = n_edges // _GCH
    niter = (nchunks + _NW - 1) // _NW
    ntrip = (niter + 2) // 3

    @functools.partial(
        pl.kernel,
        out_type=jax.ShapeDtypeStruct((n_edges, _D), jnp.float32),
        mesh=mesh,
        scratch_types=[
            pltpu.VMEM((3, _GCH), jnp.int32),
            pltpu.VMEM((3, _GCH), jnp.int32),
            pltpu.VMEM((3, _GCH, _D), jnp.float32),
            pltpu.VMEM((3, _GCH, _D), jnp.float32),
            pltpu.VMEM_SHARED((_N, _D), jnp.float32),
            pltpu.SemaphoreType.DMA,
            pltpu.SemaphoreType.DMA,
            pltpu.SemaphoreType.DMA,
        ],
    )
    def k(ta_r, tb_r, si_r, ri_r, g_out, idxs, idxr, buf_a, buf_b, stab,
          sem_i, sem_g, sem_o):
        sub = lax.axis_index("s")
        wid = lax.axis_index("c") * 16 + sub
        zrows, nzchunks = 80, _N // 80

        def stage(step, carry):
            cid = sub + step * 16

            @pl.when(cid < nzchunks)
            def _():
                sl = pl.ds(cid * zrows, zrows)
                pltpu.sync_copy(ta_r.at[sl], stab.at[sl])

            return carry

        lax.fori_loop(0, (nzchunks + 15) // 16, stage, None)
        plsc.subcore_barrier()

        def fetch(cid, slot):
            ca = pltpu.async_copy(stab.at[idxs.at[slot]], buf_a.at[slot],
                                  sem_g)
            ca.wait()
            pltpu.async_copy(tb_r.at[idxr.at[slot]], buf_b.at[slot], sem_g)

        def fire_idx(cid, slot):
            base = cid * _GCH
            pltpu.async_copy(si_r.at[pl.ds(base, _GCH)], idxs.at[slot], sem_i)
            pltpu.async_copy(ri_r.at[pl.ds(base, _GCH)], idxr.at[slot], sem_i)

        def drain_idx(slot):
            pltpu.make_async_copy(si_r.at[pl.ds(0, _GCH)], idxs.at[slot],
                                  sem_i).wait()
            pltpu.make_async_copy(ri_r.at[pl.ds(0, _GCH)], idxr.at[slot],
                                  sem_i).wait()

        def drain_out(slot):
            pltpu.make_async_copy(buf_a.at[slot], g_out.at[pl.ds(0, _GCH)],
                                  sem_o).wait()

        @pl.when(wid < nchunks)
        def _():
            base = wid * _GCH
            pltpu.sync_copy(si_r.at[pl.ds(base, _GCH)], idxs.at[0])
            pltpu.sync_copy(ri_r.at[pl.ds(base, _GCH)], idxr.at[0])
            fetch(wid, 0)

        @pl.when(wid + _NW < nchunks)
        def _():
            fire_idx(wid + _NW, 1)

        def body(step, carry):
            for u in range(3):
                s, o, oo = u, (u + 1) % 3, (u + 2) % 3
                cid = wid + (3 * step + u) * _NW
                nxt = cid + _NW

                @pl.when(cid < nchunks)
                def _(cid=cid, nxt=nxt, s=s, o=o, oo=oo):
                    pltpu.make_async_copy(tb_r.at[pl.ds(0, _GCH)],
                                          buf_b.at[s], sem_g).wait()

                    @pl.when(nxt < nchunks)
                    def _():
                        drain_idx(o)

                        @pl.when(cid >= wid + 2 * _NW)
                        def _():
                            drain_out(o)

                        fetch(nxt, o)

                    @pl.when(nxt + _NW < nchunks)
                    def _():
                        fire_idx(nxt + _NW, oo)

                    def rbody(r, c2):
                        for g in range(_D // 16):
                            sl = pl.ds(g * 16, 16)
                            buf_a[s, r, sl] = buf_a[s, r, sl] + buf_b[s, r, sl]
                        return c2

                    lax.fori_loop(0, _GCH, rbody, None)
                    pltpu.async_copy(buf_a.at[s],
                                     g_out.at[pl.ds(cid * _GCH, _GCH)], sem_o)

            return carry

        lax.fori_loop(0, ntrip, body, None)

        for q in range(3):
            @pl.when(wid + q * _NW < nchunks)
            def _(q=q):
                drain_out(q)

    return k(ta, tb, sidx, ridx)


def _edge_mlp(gsum, ef, w1e, b1, w2, b2, ln_g, ln_b):
    n_edges = gsum.shape[0]
    blk = 2000
    grid = n_edges // blk

    def body(gs_r, ef_r, w1_r, b1_r, w2_r, b2_r, g_r, be_r,
             new_r, out_r):
        ef = ef_r[...]
        pre = (gs_r[...]
               + jnp.dot(ef.astype(jnp.bfloat16),
                         w1_r[...].astype(jnp.bfloat16),
                         preferred_element_type=jnp.float32)
               + b1_r[...])
        h = jnp.maximum(pre, 0.0)
        o = jnp.dot(h.astype(jnp.bfloat16), w2_r[...].astype(jnp.bfloat16),
                    preferred_element_type=jnp.float32) + b2_r[...]
        mu = jnp.mean(o, axis=-1, keepdims=True)
        var = jnp.mean((o - mu) ** 2, axis=-1, keepdims=True)
        ln = (o - mu) * lax.rsqrt(var + 1e-5) * g_r[...] + be_r[...]
        new_r[...] = ln
        out_r[...] = ln + ef

    row = pl.BlockSpec((blk, _D), lambda i: (i, 0))
    full = pl.BlockSpec((_D, _D), lambda i: (0, 0))
    vec = pl.BlockSpec((1, _D), lambda i: (0, 0))
    return pl.pallas_call(
        body,
        grid=(grid,),
        in_specs=[row, row, full, vec, full, vec, vec, vec],
        out_specs=[row, row],
        out_shape=[jax.ShapeDtypeStruct((n_edges, _D), jnp.float32)] * 2,
    )(gsum, ef, w1e, b1.reshape(1, _D), w2, b2.reshape(1, _D),
      ln_g.reshape(1, _D), ln_b.reshape(1, _D))


def _sc_scatter(new_e, ridx, n_edges):
    mesh = plsc.VectorSubcoreMesh(core_axis_name="c", subcore_axis_name="s")
    zrows = 80
    nzchunks = _N // zrows

    @functools.partial(
        pl.kernel,
        out_type=jax.ShapeDtypeStruct((2 * _N, _D), jnp.float32),
        mesh=mesh,
        scratch_types=[
            pltpu.VMEM((zrows, _D), jnp.float32),
            pltpu.VMEM((2, 128), jnp.int32),
            pltpu.VMEM((_CH, _D), jnp.float32),
            pltpu.VMEM_SHARED((_N, _D), jnp.float32),
            pltpu.SemaphoreType.DMA,
        ],
    )
    def k(src, idx, out, zbuf, idx_v, rows_v, acc, sem):
        core = lax.axis_index("c")
        sub = lax.axis_index("s")
        wid = core * 16 + sub

        zero16 = jnp.zeros((16,), jnp.float32)

        def zbody(i, carry):
            for j in range(_D // 16):
                zbuf[i, pl.ds(j * 16, 16)] = zero16
            return carry

        lax.fori_loop(0, zrows, zbody, None)

        def zero_acc(step, carry):
            cid = sub + step * 16

            @pl.when(cid < nzchunks)
            def _():
                pltpu.sync_copy(zbuf, acc.at[pl.ds(cid * zrows, zrows)])

            return carry

        lax.fori_loop(0, (nzchunks + 15) // 16, zero_acc, None)
        plsc.subcore_barrier()

        nchunks = n_edges // _CH
        niter = (nchunks + _NW - 1) // _NW

        def body(step, carry):
            cid = wid + step * _NW

            @pl.when(cid < nchunks)
            def _():
                base = cid * _CH
                pltpu.sync_copy(idx.at[pl.ds(base, 128)], idx_v.at[0])
                pltpu.sync_copy(idx.at[pl.ds(base + 128, 128)], idx_v.at[1])
                pltpu.sync_copy(src.at[pl.ds(base, _CH)], rows_v)
                pltpu.sync_copy(rows_v.at[pl.ds(0, 128)],
                                acc.at[idx_v.at[0]], add=True)
                pltpu.sync_copy(rows_v.at[pl.ds(128, 128)],
                                acc.at[idx_v.at[1]], add=True)

            return carry

        lax.fori_loop(0, niter, body, None)
        plsc.subcore_barrier()

        def writeout(step, carry):
            cid = sub + step * 16

            @pl.when(cid < nzchunks)
            def _():
                pltpu.sync_copy(
                    acc.at[pl.ds(cid * zrows, zrows)],
                    out.at[pl.ds(core * _N + cid * zrows, zrows)])

            return carry

        lax.fori_loop(0, (nzchunks + 15) // 16, writeout, None)
        plsc.subcore_barrier()

    return k(new_e, ridx)


def _node_mlp(nf, am0, am1, aw0, aw1, w_stack, b1, w2, b2, ln_g, ln_b):
    blk = 1000
    grid = _N // blk

    def body(nf_r, am0_r, am1_r, aw0_r, aw1_r, w_r, b1_r, w2_r, b2_r,
             g_r, be_r, out_r):
        nfx = nf_r[...]
        agg_m = am0_r[...] + am1_r[...]
        agg_w = aw0_r[...] + aw1_r[...]
        w16 = w_r[...].astype(jnp.bfloat16)
        pre = (jnp.dot(nfx.astype(jnp.bfloat16), w16[0],
                       preferred_element_type=jnp.float32)
               + jnp.dot(agg_m.astype(jnp.bfloat16), w16[1],
                         preferred_element_type=jnp.float32)
               + jnp.dot(agg_w.astype(jnp.bfloat16), w16[2],
                         preferred_element_type=jnp.float32)
               + b1_r[...])
        h = jnp.maximum(pre, 0.0)
        o = jnp.dot(h.astype(jnp.bfloat16), w2_r[...].astype(jnp.bfloat16),
                    preferred_element_type=jnp.float32) + b2_r[...]
        mu = jnp.mean(o, axis=-1, keepdims=True)
        var = jnp.mean((o - mu) ** 2, axis=-1, keepdims=True)
        ln = (o - mu) * lax.rsqrt(var + 1e-5) * g_r[...] + be_r[...]
        out_r[...] = ln + nfx

    row = pl.BlockSpec((blk, _D), lambda i: (i, 0))
    full = pl.BlockSpec((_D, _D), lambda i: (0, 0))
    vec = pl.BlockSpec((1, _D), lambda i: (0, 0))
    return pl.pallas_call(
        body,
        grid=(grid,),
        in_specs=[row, row, row, row, row,
                  pl.BlockSpec((3, _D, _D), lambda i: (0, 0, 0)),
                  vec, full, vec, vec, vec],
        out_specs=row,
        out_shape=jax.ShapeDtypeStruct((_N, _D), jnp.float32),
    )(nf, am0, am1, aw0, aw1, w_stack, b1.reshape(1, _D), w2,
      b2.reshape(1, _D), ln_g.reshape(1, _D), ln_b.reshape(1, _D))


def kernel(node_features, mesh_edge_features, mesh_senders, mesh_receivers,
           world_edge_features, world_senders, world_receivers, params):
    pm, pw, pn = params["mesh_edge"], params["world_edge"], params["node"]

    w_gather = jnp.stack([pm["W1"][:_D], pm["W1"][_D:2 * _D],
                          pw["W1"][:_D], pw["W1"][_D:2 * _D]])
    tam, tbm, taw, tbw = _precompute_tables(node_features, w_gather)

    ms = mesh_senders.astype(jnp.int32)
    mr = mesh_receivers.astype(jnp.int32)
    ws = world_senders.astype(jnp.int32)
    wr = world_receivers.astype(jnp.int32)

    gm = _sc_gather(tam, tbm, ms, mr, _E_MESH)
    gw = _sc_gather(taw, tbw, ws, wr, _E_WORLD)

    new_m, out_m = _edge_mlp(gm, mesh_edge_features,
                             pm["W1"][2 * _D:], pm["b1"], pm["W2"], pm["b2"],
                             pm["ln_g"], pm["ln_b"])
    new_w, out_w = _edge_mlp(gw, world_edge_features,
                             pw["W1"][2 * _D:], pw["b1"], pw["W2"], pw["b2"],
                             pw["ln_g"], pw["ln_b"])

    aggm2 = _sc_scatter(new_m, mr, _E_MESH)
    aggw2 = _sc_scatter(new_w, wr, _E_WORLD)

    w_node = jnp.stack([pn["W1"][:_D], pn["W1"][_D:2 * _D], pn["W1"][2 * _D:]])
    new_nodes = _node_mlp(node_features,
                          aggm2[:_N], aggm2[_N:], aggw2[:_N], aggw2[_N:],
                          w_node, pn["b1"], pn["W2"], pn["b2"],
                          pn["ln_g"], pn["ln_b"])
    return new_nodes, out_m, out_w


# speedup vs baseline: 1.5349x; 1.0797x over previous
import functools

import jax
import jax.numpy as jnp
from jax import lax
from jax.experimental import pallas as pl
from jax.experimental.pallas import tpu as pltpu
from jax.experimental.pallas import tpu_sc as plsc

_N = 10000
_D = 128
_E_MESH = 320000
_E_WORLD = 160000
_NW = 32
_CH = 256


def _precompute_tables(nf, w_stack):
    blk = 2000
    grid = _N // blk

    def body(nf_ref, w_ref, oa_m, ob_m, oa_w, ob_w):
        x = nf_ref[...].astype(jnp.bfloat16)
        for j, o_ref in enumerate((oa_m, ob_m, oa_w, ob_w)):
            o_ref[...] = jnp.dot(x, w_ref[j].astype(jnp.bfloat16),
                                 preferred_element_type=jnp.float32)

    out = pl.pallas_call(
        body,
        grid=(grid,),
        in_specs=[
            pl.BlockSpec((blk, _D), lambda i: (i, 0)),
            pl.BlockSpec((4, _D, _D), lambda i: (0, 0, 0)),
        ],
        out_specs=[pl.BlockSpec((blk, _D), lambda i: (i, 0))] * 4,
        out_shape=[jax.ShapeDtypeStruct((_N, _D), jnp.float32)] * 4,
    )(nf, w_stack)
    return out


_GCH = 64


def _sc_gather(ta, tb, sidx, ridx, n_edges):
    mesh = plsc.VectorSubcoreMesh(core_axis_name="c", subcore_axis_name="s")
    nchunks = n_edges // _GCH
    niter = (nchunks + _NW - 1) // _NW
    ntrip = (niter + 2) // 3

    @functools.partial(
        pl.kernel,
        out_type=jax.ShapeDtypeStruct((n_edges, _D), jnp.float32),
        mesh=mesh,
        scratch_types=[
            pltpu.VMEM((3, _GCH), jnp.int32),
            pltpu.VMEM((3, _GCH), jnp.int32),
            pltpu.VMEM((3, _GCH, _D), jnp.float32),
            pltpu.VMEM((3, _GCH, _D), jnp.float32),
            pltpu.VMEM_SHARED((_N, _D), jnp.float32),
            pltpu.SemaphoreType.DMA,
            pltpu.SemaphoreType.DMA,
            pltpu.SemaphoreType.DMA,
        ],
    )
    def k(ta_r, tb_r, si_r, ri_r, g_out, idxs, idxr, buf_a, buf_b, stab,
          sem_i, sem_g, sem_o):
        sub = lax.axis_index("s")
        wid = lax.axis_index("c") * 16 + sub
        zrows, nzchunks = 80, _N // 80

        def stage(step, carry):
            cid = sub + step * 16

            @pl.when(cid < nzchunks)
            def _():
                sl = pl.ds(cid * zrows, zrows)
                pltpu.sync_copy(ta_r.at[sl], stab.at[sl])

            return carry

        lax.fori_loop(0, (nzchunks + 15) // 16, stage, None)
        plsc.subcore_barrier()

        def fetch(cid, slot):
            ca = pltpu.async_copy(stab.at[idxs.at[slot]], buf_a.at[slot],
                                  sem_g)
            ca.wait()
            pltpu.async_copy(tb_r.at[idxr.at[slot]], buf_b.at[slot], sem_g)

        def fire_idx(cid, slot):
            base = cid * _GCH
            pltpu.async_copy(si_r.at[pl.ds(base, _GCH)], idxs.at[slot], sem_i)
            pltpu.async_copy(ri_r.at[pl.ds(base, _GCH)], idxr.at[slot], sem_i)

        def drain_idx(slot):
            pltpu.make_async_copy(si_r.at[pl.ds(0, _GCH)], idxs.at[slot],
                                  sem_i).wait()
            pltpu.make_async_copy(ri_r.at[pl.ds(0, _GCH)], idxr.at[slot],
                                  sem_i).wait()

        def drain_out(slot):
            pltpu.make_async_copy(buf_a.at[slot], g_out.at[pl.ds(0, _GCH)],
                                  sem_o).wait()

        @pl.when(wid < nchunks)
        def _():
            base = wid * _GCH
            pltpu.sync_copy(si_r.at[pl.ds(base, _GCH)], idxs.at[0])
            pltpu.sync_copy(ri_r.at[pl.ds(base, _GCH)], idxr.at[0])
            fetch(wid, 0)

        @pl.when(wid + _NW < nchunks)
        def _():
            fire_idx(wid + _NW, 1)

        def body(step, carry):
            for u in range(3):
                s, o, oo = u, (u + 1) % 3, (u + 2) % 3
                cid = wid + (3 * step + u) * _NW
                nxt = cid + _NW

                @pl.when(cid < nchunks)
                def _(cid=cid, nxt=nxt, s=s, o=o, oo=oo):
                    pltpu.make_async_copy(tb_r.at[pl.ds(0, _GCH)],
                                          buf_b.at[s], sem_g).wait()

                    @pl.when(nxt < nchunks)
                    def _():
                        drain_idx(o)

                        @pl.when(cid >= wid + 2 * _NW)
                        def _():
                            drain_out(o)

                        fetch(nxt, o)

                    @pl.when(nxt + _NW < nchunks)
                    def _():
                        fire_idx(nxt + _NW, oo)

                    def rbody(r, c2):
                        for g in range(_D // 16):
                            sl = pl.ds(g * 16, 16)
                            buf_a[s, r, sl] = buf_a[s, r, sl] + buf_b[s, r, sl]
                        return c2

                    lax.fori_loop(0, _GCH, rbody, None)
                    pltpu.async_copy(buf_a.at[s],
                                     g_out.at[pl.ds(cid * _GCH, _GCH)], sem_o)

            return carry

        lax.fori_loop(0, ntrip, body, None)

        for q in range(3):
            @pl.when(wid + q * _NW < nchunks)
            def _(q=q):
                drain_out(q)

    return k(ta, tb, sidx, ridx)


def _edge_mlp(gsum, ef, w1e, b1, w2, b2, ln_g, ln_b):
    n_edges = gsum.shape[0]
    blk = 2000
    grid = n_edges // blk

    def body(gs_r, ef_r, w1_r, b1_r, w2_r, b2_r, g_r, be_r,
             new_r, out_r):
        ef = ef_r[...]
        pre = (gs_r[...]
               + jnp.dot(ef.astype(jnp.bfloat16),
                         w1_r[...].astype(jnp.bfloat16),
                         preferred_element_type=jnp.float32)
               + b1_r[...])
        h = jnp.maximum(pre, 0.0)
        o = jnp.dot(h.astype(jnp.bfloat16), w2_r[...].astype(jnp.bfloat16),
                    preferred_element_type=jnp.float32) + b2_r[...]
        mu = jnp.mean(o, axis=-1, keepdims=True)
        var = jnp.mean((o - mu) ** 2, axis=-1, keepdims=True)
        ln = (o - mu) * lax.rsqrt(var + 1e-5) * g_r[...] + be_r[...]
        new_r[...] = ln
        out_r[...] = ln + ef

    row = pl.BlockSpec((blk, _D), lambda i: (i, 0))
    full = pl.BlockSpec((_D, _D), lambda i: (0, 0))
    vec = pl.BlockSpec((1, _D), lambda i: (0, 0))
    return pl.pallas_call(
        body,
        grid=(grid,),
        in_specs=[row, row, full, vec, full, vec, vec, vec],
        out_specs=[row, row],
        out_shape=[jax.ShapeDtypeStruct((n_edges, _D), jnp.float32)] * 2,
    )(gsum, ef, w1e, b1.reshape(1, _D), w2, b2.reshape(1, _D),
      ln_g.reshape(1, _D), ln_b.reshape(1, _D))


_SCH = 128


def _sc_scatter(new_e, ridx, n_edges):
    mesh = plsc.VectorSubcoreMesh(core_axis_name="c", subcore_axis_name="s")
    zrows = 80
    nzchunks = _N // zrows

    @functools.partial(
        pl.kernel,
        out_type=jax.ShapeDtypeStruct((2 * _N, _D), jnp.float32),
        mesh=mesh,
        scratch_types=[
            pltpu.VMEM((zrows, _D), jnp.float32),
            pltpu.VMEM((2, _SCH), jnp.int32),
            pltpu.VMEM((2, _SCH, _D), jnp.float32),
            pltpu.VMEM_SHARED((_N, _D), jnp.float32),
            pltpu.SemaphoreType.DMA,
            pltpu.SemaphoreType.DMA,
        ],
    )
    def k(src, idx, out, zbuf, idx_v, rows_v, acc, sem_i, sem_r):
        core = lax.axis_index("c")
        sub = lax.axis_index("s")
        wid = core * 16 + sub

        zero16 = jnp.zeros((16,), jnp.float32)

        def zbody(i, carry):
            for j in range(_D // 16):
                zbuf[i, pl.ds(j * 16, 16)] = zero16
            return carry

        lax.fori_loop(0, zrows, zbody, None)

        def zero_acc(step, carry):
            cid = sub + step * 16

            @pl.when(cid < nzchunks)
            def _():
                pltpu.sync_copy(zbuf, acc.at[pl.ds(cid * zrows, zrows)])

            return carry

        lax.fori_loop(0, (nzchunks + 15) // 16, zero_acc, None)
        plsc.subcore_barrier()

        nchunks = n_edges // _SCH
        niter = (nchunks + _NW - 1) // _NW
        npairs = (niter + 1) // 2

        def fire_in(cid, slot):
            base = cid * _SCH
            pltpu.async_copy(idx.at[pl.ds(base, _SCH)], idx_v.at[slot], sem_i)
            pltpu.async_copy(src.at[pl.ds(base, _SCH)], rows_v.at[slot],
                             sem_r)

        @pl.when(wid < nchunks)
        def _():
            fire_in(wid, 0)

        def body(step, carry):
            for u in range(2):
                s, o = (0, 1) if u == 0 else (1, 0)
                cid = wid + (2 * step + u) * _NW
                nxt = cid + _NW

                @pl.when(cid < nchunks)
                def _(cid=cid, nxt=nxt, s=s, o=o):
                    pltpu.make_async_copy(idx.at[pl.ds(0, _SCH)],
                                          idx_v.at[s], sem_i).wait()
                    pltpu.make_async_copy(src.at[pl.ds(0, _SCH)],
                                          rows_v.at[s], sem_r).wait()

                    @pl.when(nxt < nchunks)
                    def _():
                        fire_in(nxt, o)

                    pltpu.sync_copy(rows_v.at[s], acc.at[idx_v.at[s]],
                                    add=True)

            return carry

        lax.fori_loop(0, npairs, body, None)
        plsc.subcore_barrier()

        def writeout(step, carry):
            cid = sub + step * 16

            @pl.when(cid < nzchunks)
            def _():
                pltpu.sync_copy(
                    acc.at[pl.ds(cid * zrows, zrows)],
                    out.at[pl.ds(core * _N + cid * zrows, zrows)])

            return carry

        lax.fori_loop(0, (nzchunks + 15) // 16, writeout, None)
        plsc.subcore_barrier()

    return k(new_e, ridx)


def _node_mlp(nf, am0, am1, aw0, aw1, w_stack, b1, w2, b2, ln_g, ln_b):
    blk = 1000
    grid = _N // blk

    def body(nf_r, am0_r, am1_r, aw0_r, aw1_r, w_r, b1_r, w2_r, b2_r,
             g_r, be_r, out_r):
        nfx = nf_r[...]
        agg_m = am0_r[...] + am1_r[...]
        agg_w = aw0_r[...] + aw1_r[...]
        w16 = w_r[...].astype(jnp.bfloat16)
        pre = (jnp.dot(nfx.astype(jnp.bfloat16), w16[0],
                       preferred_element_type=jnp.float32)
               + jnp.dot(agg_m.astype(jnp.bfloat16), w16[1],
                         preferred_element_type=jnp.float32)
               + jnp.dot(agg_w.astype(jnp.bfloat16), w16[2],
                         preferred_element_type=jnp.float32)
               + b1_r[...])
        h = jnp.maximum(pre, 0.0)
        o = jnp.dot(h.astype(jnp.bfloat16), w2_r[...].astype(jnp.bfloat16),
                    preferred_element_type=jnp.float32) + b2_r[...]
        mu = jnp.mean(o, axis=-1, keepdims=True)
        var = jnp.mean((o - mu) ** 2, axis=-1, keepdims=True)
        ln = (o - mu) * lax.rsqrt(var + 1e-5) * g_r[...] + be_r[...]
        out_r[...] = ln + nfx

    row = pl.BlockSpec((blk, _D), lambda i: (i, 0))
    full = pl.BlockSpec((_D, _D), lambda i: (0, 0))
    vec = pl.BlockSpec((1, _D), lambda i: (0, 0))
    return pl.pallas_call(
        body,
        grid=(grid,),
        in_specs=[row, row, row, row, row,
                  pl.BlockSpec((3, _D, _D), lambda i: (0, 0, 0)),
                  vec, full, vec, vec, vec],
        out_specs=row,
        out_shape=jax.ShapeDtypeStruct((_N, _D), jnp.float32),
    )(nf, am0, am1, aw0, aw1, w_stack, b1.reshape(1, _D), w2,
      b2.reshape(1, _D), ln_g.reshape(1, _D), ln_b.reshape(1, _D))


def kernel(node_features, mesh_edge_features, mesh_senders, mesh_receivers,
           world_edge_features, world_senders, world_receivers, params):
    pm, pw, pn = params["mesh_edge"], params["world_edge"], params["node"]

    w_gather = jnp.stack([pm["W1"][:_D], pm["W1"][_D:2 * _D],
                          pw["W1"][:_D], pw["W1"][_D:2 * _D]])
    tam, tbm, taw, tbw = _precompute_tables(node_features, w_gather)

    ms = mesh_senders.astype(jnp.int32)
    mr = mesh_receivers.astype(jnp.int32)
    ws = world_senders.astype(jnp.int32)
    wr = world_receivers.astype(jnp.int32)

    gm = _sc_gather(tam, tbm, ms, mr, _E_MESH)
    gw = _sc_gather(taw, tbw, ws, wr, _E_WORLD)

    new_m, out_m = _edge_mlp(gm, mesh_edge_features,
                             pm["W1"][2 * _D:], pm["b1"], pm["W2"], pm["b2"],
                             pm["ln_g"], pm["ln_b"])
    new_w, out_w = _edge_mlp(gw, world_edge_features,
                             pw["W1"][2 * _D:], pw["b1"], pw["W2"], pw["b2"],
                             pw["ln_g"], pw["ln_b"])

    aggm2 = _sc_scatter(new_m, mr, _E_MESH)
    aggw2 = _sc_scatter(new_w, wr, _E_WORLD)

    w_node = jnp.stack([pn["W1"][:_D], pn["W1"][_D:2 * _D], pn["W1"][2 * _D:]])
    new_nodes = _node_mlp(node_features,
                          aggm2[:_N], aggm2[_N:], aggw2[:_N], aggw2[_N:],
                          w_node, pn["b1"], pn["W2"], pn["b2"],
                          pn["ln_g"], pn["ln_b"])
    return new_nodes, out_m, out_w
